# Initial kernel scaffold; baseline (speedup 1.0000x reference)
import jax
import jax.numpy as jnp
from jax.experimental import pallas as pl


def kernel(h, e, edge_index, edge_w1, edge_b1, edge_w2, edge_b2, gate_w1, gate_b1, gate_w2, gate_b2, node_w1, node_b1, node_w2, node_b2):
    raise NotImplementedError("write your pallas kernel here")


# speedup vs baseline: 2.8082x; 2.8082x over previous
import functools

import jax
import jax.numpy as jnp
from jax import lax
from jax.experimental import pallas as pl
from jax.experimental.pallas import tpu as pltpu
from jax.experimental.pallas import tpu_sc as plsc

NC = 2
NS = 16
NW = NC * NS
CHUNK = 128


def _sc_mesh():
    return plsc.VectorSubcoreMesh(core_axis_name="c", subcore_axis_name="s")


def _make_gather(E, N, H):
    epw = E // NW
    nfull = epw // CHUNK
    tail = epw - nfull * CHUNK
    sds = jax.ShapeDtypeStruct((E, H), jnp.float32)

    @functools.partial(
        pl.kernel,
        mesh=_sc_mesh(),
        out_type=(sds, sds),
        scratch_types=[
            pltpu.VMEM((CHUNK,), jnp.int32),
            pltpu.VMEM((CHUNK,), jnp.int32),
            pltpu.VMEM((CHUNK, H), jnp.float32),
            pltpu.VMEM((CHUNK, H), jnp.float32),
            pltpu.VMEM((tail,), jnp.int32),
            pltpu.VMEM((tail,), jnp.int32),
            pltpu.VMEM((tail, H), jnp.float32),
            pltpu.VMEM((tail, H), jnp.float32),
            pltpu.SemaphoreType.DMA,
            pltpu.SemaphoreType.DMA,
        ],
    )
    def gather_kernel(ta, tb, row, col, outa, outb,
                      idxa, idxb, bufa, bufb, tidxa, tidxb, tbufa, tbufb,
                      sema, semb):
        wid = lax.axis_index("s") * NC + lax.axis_index("c")
        base = wid * epw

        @pl.loop(0, nfull)
        def _(j):
            off = base + j * CHUNK
            pltpu.sync_copy(row.at[pl.ds(off, CHUNK)], idxa)
            pltpu.sync_copy(col.at[pl.ds(off, CHUNK)], idxb)
            ca = pltpu.async_copy(ta.at[idxa], bufa, sema)
            cb = pltpu.async_copy(tb.at[idxb], bufb, semb)
            ca.wait()
            cb.wait()
            pltpu.sync_copy(bufa, outa.at[pl.ds(off, CHUNK)])
            pltpu.sync_copy(bufb, outb.at[pl.ds(off, CHUNK)])

        if tail:
            off = base + nfull * CHUNK
            pltpu.sync_copy(row.at[pl.ds(off, tail)], tidxa)
            pltpu.sync_copy(col.at[pl.ds(off, tail)], tidxb)
            ca = pltpu.async_copy(ta.at[tidxa], tbufa, sema)
            cb = pltpu.async_copy(tb.at[tidxb], tbufb, semb)
            ca.wait()
            cb.wait()
            pltpu.sync_copy(tbufa, outa.at[pl.ds(off, tail)])
            pltpu.sync_copy(tbufb, outb.at[pl.ds(off, tail)])

    return gather_kernel


def _make_scatter(E, N, H):
    epw = E // NW
    nfull = epw // CHUNK
    tail = epw - nfull * CHUNK
    rps = (N // NS) & ~7
    rrem = N - NS * rps

    @functools.partial(
        pl.kernel,
        mesh=_sc_mesh(),
        out_type=jax.ShapeDtypeStruct((NC, N, H), jnp.float32),
        scratch_types=[
            pltpu.VMEM((CHUNK,), jnp.int32),
            pltpu.VMEM((CHUNK, H), jnp.float32),
            pltpu.VMEM((tail,), jnp.int32),
            pltpu.VMEM((tail, H), jnp.float32),
            pltpu.VMEM_SHARED((N, H), jnp.float32),
        ],
    )
    def scatter_kernel(e_h, row_h, zeros_h, out_h, idx, buf, tidx, tbuf, agg):
        cid = lax.axis_index("c")
        sid = lax.axis_index("s")
        wid = sid * NC + cid
        rbase = sid * rps
        pltpu.sync_copy(zeros_h.at[pl.ds(rbase, rps)], agg.at[pl.ds(rbase, rps)])
        if rrem:
            @pl.when(sid == 0)
            def _():
                pltpu.sync_copy(zeros_h.at[pl.ds(NS * rps, rrem)],
                                agg.at[pl.ds(NS * rps, rrem)])
        plsc.subcore_barrier()

        base = wid * epw

        @pl.loop(0, nfull)
        def _(j):
            off = base + j * CHUNK
            pltpu.sync_copy(row_h.at[pl.ds(off, CHUNK)], idx)
            pltpu.sync_copy(e_h.at[pl.ds(off, CHUNK)], buf)
            pltpu.sync_copy(buf, agg.at[idx], add=True)

        if tail:
            off = base + nfull * CHUNK
            pltpu.sync_copy(row_h.at[pl.ds(off, tail)], tidx)
            pltpu.sync_copy(e_h.at[pl.ds(off, tail)], tbuf)
            pltpu.sync_copy(tbuf, agg.at[tidx], add=True)

        plsc.subcore_barrier()
        pltpu.sync_copy(agg.at[pl.ds(rbase, rps)], out_h.at[cid, pl.ds(rbase, rps)])
        if rrem:
            @pl.when(sid == 0)
            def _():
                pltpu.sync_copy(agg.at[pl.ds(NS * rps, rrem)],
                                out_h.at[cid, pl.ds(NS * rps, rrem)])

    return scatter_kernel


def _edge_body(ga_ref, gb_ref, e_ref, w1c_ref, w2_ref, b2_ref, out_ref):
    x = ga_ref[...] + gb_ref[...] + jnp.dot(
        e_ref[...], w1c_ref[...], preferred_element_type=jnp.float32)
    x = jnp.maximum(x, 0.0)
    out_ref[...] = jnp.dot(
        x, w2_ref[...], preferred_element_type=jnp.float32) + (e_ref[...] + b2_ref[...])


def _proj_body(h_ref, w1a_ref, w1b_ref, b1_ref, oa_ref, ob_ref):
    hv = h_ref[...]
    oa_ref[...] = jnp.dot(
        hv, w1a_ref[...], preferred_element_type=jnp.float32) + b1_ref[...]
    ob_ref[...] = jnp.dot(hv, w1b_ref[...], preferred_element_type=jnp.float32)


def _node_body(h_ref, a0_ref, a1_ref, w1a_ref, w1b_ref, b1_ref, w2_ref, b2_ref,
               out_ref):
    agg = a0_ref[...] + a1_ref[...]
    x = (jnp.dot(h_ref[...], w1a_ref[...], preferred_element_type=jnp.float32)
         + jnp.dot(agg, w1b_ref[...], preferred_element_type=jnp.float32)
         + b1_ref[...])
    x = jnp.maximum(x, 0.0)
    out_ref[...] = jnp.dot(
        x, w2_ref[...], preferred_element_type=jnp.float32) + b2_ref[...] + h_ref[...]


def kernel(h, e, edge_index, edge_w1, edge_b1, edge_w2, edge_b2,
           gate_w1, gate_b1, gate_w2, gate_b2,
           node_w1, node_b1, node_w2, node_b2):
    N, H = h.shape
    E = e.shape[0]
    assert E % (NW * 8) == 0 and N % NS == 0

    row = edge_index[0]
    col = edge_index[1]
    w1a, w1b, w1c = edge_w1[:H], edge_w1[H:2 * H], edge_w1[2 * H:]
    b1 = edge_b1.reshape(1, H)
    b2 = edge_b2.reshape(1, H)

    PT = 1000
    proj = pl.pallas_call(
        _proj_body,
        grid=(N // PT,),
        in_specs=[
            pl.BlockSpec((PT, H), lambda i: (i, 0)),
            pl.BlockSpec((H, H), lambda i: (0, 0)),
            pl.BlockSpec((H, H), lambda i: (0, 0)),
            pl.BlockSpec((1, H), lambda i: (0, 0)),
        ],
        out_specs=[
            pl.BlockSpec((PT, H), lambda i: (i, 0)),
            pl.BlockSpec((PT, H), lambda i: (i, 0)),
        ],
        out_shape=[
            jax.ShapeDtypeStruct((N, H), jnp.float32),
            jax.ShapeDtypeStruct((N, H), jnp.float32),
        ],
    )
    ha, hb = proj(h, w1a, w1b, b1)

    ga, gb = _make_gather(E, N, H)(ha, hb, row, col)

    aggp = _make_scatter(E, N, H)(e, row, jnp.zeros((N, H), jnp.float32))

    ET = 2000
    edge_feat = pl.pallas_call(
        _edge_body,
        grid=(E // ET,),
        in_specs=[
            pl.BlockSpec((ET, H), lambda i: (i, 0)),
            pl.BlockSpec((ET, H), lambda i: (i, 0)),
            pl.BlockSpec((ET, H), lambda i: (i, 0)),
            pl.BlockSpec((H, H), lambda i: (0, 0)),
            pl.BlockSpec((H, H), lambda i: (0, 0)),
            pl.BlockSpec((1, H), lambda i: (0, 0)),
        ],
        out_specs=pl.BlockSpec((ET, H), lambda i: (i, 0)),
        out_shape=jax.ShapeDtypeStruct((E, H), jnp.float32),
    )(ga, gb, e, w1c, edge_w2, b2)

    nw1a, nw1b = node_w1[:H], node_w1[H:]
    NT = 1000
    h_out = pl.pallas_call(
        _node_body,
        grid=(N // NT,),
        in_specs=[
            pl.BlockSpec((NT, H), lambda i: (i, 0)),
            pl.BlockSpec((NT, H), lambda i: (i, 0)),
            pl.BlockSpec((NT, H), lambda i: (i, 0)),
            pl.BlockSpec((H, H), lambda i: (0, 0)),
            pl.BlockSpec((H, H), lambda i: (0, 0)),
            pl.BlockSpec((1, H), lambda i: (0, 0)),
            pl.BlockSpec((H, H), lambda i: (0, 0)),
            pl.BlockSpec((1, H), lambda i: (0, 0)),
        ],
        out_specs=pl.BlockSpec((NT, H), lambda i: (i, 0)),
        out_shape=jax.ShapeDtypeStruct((N, H), jnp.float32),
    )(h, aggp[0], aggp[1], nw1a, nw1b, node_b1.reshape(1, H), node_w2,
      node_b2.reshape(1, H))

    return (h_out, edge_feat)
